# baseline (speedup 1.0000x reference)
import jax
import jax.numpy as jnp
from jax.experimental import pallas as pl
from jax.experimental.pallas import tpu as pltpu

_N_E = 1024
_E_DIM = 64
_DEPTH = 6
_BM = 512


def _vq_block(zf_ref, sw_ref, tw_ref, zqs_ref, zqt_ref, inds_ref, indt_ref):
    x = zf_ref[...]
    col = jax.lax.broadcasted_iota(jnp.int32, (x.shape[0], _N_E), 1)
    sw = sw_ref[...]
    tw = tw_ref[...]
    sw_sq = jnp.sum(sw ** 2, axis=1)
    tw_sq = jnp.sum(tw ** 2, axis=1)
    res_s, zq_s = x, jnp.zeros_like(x)
    res_t, zq_t = x, jnp.zeros_like(x)
    for depth in range(_DEPTH):
        def step(residual, zq, cb, cb_sq, ind_ref):
            d = (jnp.sum(residual ** 2, axis=1, keepdims=True)
                 + cb_sq
                 - 2.0 * (residual @ cb.T))
            dmin = jnp.min(d, axis=1, keepdims=True)
            mi = jnp.min(jnp.where(d <= dmin, col, _N_E), axis=1)
            onehot = (col == mi[:, None]).astype(jnp.float32)
            delta = jax.lax.dot_general(
                onehot, cb, (((1,), (0,)), ((), ())),
                precision=jax.lax.Precision.HIGHEST,
                preferred_element_type=jnp.float32)
            ind_ref[0, depth, :] = mi
            return residual - delta, zq + delta

        res_s, zq_s = step(res_s, zq_s, sw, sw_sq, inds_ref)
        res_t, zq_t = step(res_t, zq_t, tw, tw_sq, indt_ref)
    zqs_ref[...] = zq_s
    zqt_ref[...] = zq_t


def _cos_block(sw_ref, tw_ref, out_ref):
    sw = sw_ref[...]
    tw = tw_ref[...]
    sn = sw / (jnp.sqrt(jnp.sum(sw * sw, axis=1, keepdims=True)) + 1e-8)
    tn = tw / (jnp.sqrt(jnp.sum(tw * tw, axis=1, keepdims=True)) + 1e-8)
    m = jnp.dot(sn, tn.T, preferred_element_type=jnp.float32)
    out_ref[...] = (jnp.sum(m * m) / (_N_E * _N_E))[None, None]


def kernel(z, shared_w, task_w):
    b, c, h, w = z.shape
    ks = 2
    lh, lw = h - ks + 1, w - ks + 1
    pats = [z[:, :, i:i + lh, j:j + lw] for i in range(ks) for j in range(ks)]
    p = jnp.stack(pats, axis=2)
    zf = (p.reshape(b, c * ks * ks, lh * lw)
           .transpose(0, 2, 1)
           .reshape(-1, _E_DIM))
    n = zf.shape[0]
    nblk = (n + _BM - 1) // _BM
    npad = nblk * _BM
    zf_pad = jnp.pad(zf, ((0, npad - n), (0, 0)))

    zqs, zqt, inds_blk, indt_blk = pl.pallas_call(
        _vq_block,
        grid=(nblk,),
        compiler_params=pltpu.CompilerParams(
            dimension_semantics=("parallel",)),
        in_specs=[
            pl.BlockSpec((_BM, _E_DIM), lambda i: (i, 0)),
            pl.BlockSpec((_N_E, _E_DIM), lambda i: (0, 0)),
            pl.BlockSpec((_N_E, _E_DIM), lambda i: (0, 0)),
        ],
        out_specs=[
            pl.BlockSpec((_BM, _E_DIM), lambda i: (i, 0)),
            pl.BlockSpec((_BM, _E_DIM), lambda i: (i, 0)),
            pl.BlockSpec((1, _DEPTH, _BM), lambda i: (i, 0, 0)),
            pl.BlockSpec((1, _DEPTH, _BM), lambda i: (i, 0, 0)),
        ],
        out_shape=[
            jax.ShapeDtypeStruct((npad, _E_DIM), jnp.float32),
            jax.ShapeDtypeStruct((npad, _E_DIM), jnp.float32),
            jax.ShapeDtypeStruct((nblk, _DEPTH, _BM), jnp.int32),
            jax.ShapeDtypeStruct((nblk, _DEPTH, _BM), jnp.int32),
        ],
    )(zf_pad, shared_w, task_w)

    ind_s = (inds_blk.transpose(0, 2, 1).reshape(npad, _DEPTH)[:n]
             .reshape(b, lh, lw, _DEPTH))
    ind_t = (indt_blk.transpose(0, 2, 1).reshape(npad, _DEPTH)[:n]
             .reshape(b, lh, lw, _DEPTH))

    ch = jnp.where((jnp.arange(h) == 0) | (jnp.arange(h) == h - 1), 1.0, 2.0)
    cw = jnp.where((jnp.arange(w) == 0) | (jnp.arange(w) == w - 1), 1.0, 2.0)
    cnt = ch[:, None] * cw[None, :]

    def fold(zq_flat):
        zq = zq_flat[:n].reshape(b, lh, lw, c, ks * ks)
        out = jnp.zeros((b, c, h, w), jnp.float32)
        idx = 0
        for i in range(ks):
            for j in range(ks):
                out = out.at[:, :, i:i + lh, j:j + lw].add(
                    zq[..., idx].transpose(0, 3, 1, 2))
                idx += 1
        return out / cnt

    zq_s_f = fold(zqs)
    zq_t_f = fold(zqt)
    zq_out = 0.5 * (zq_s_f + zq_t_f)

    cos_loss = pl.pallas_call(
        _cos_block,
        out_shape=jax.ShapeDtypeStruct((1, 1), jnp.float32),
    )(shared_w, task_w)[0, 0]

    beta = 0.25
    loss = ((1.0 + beta) * (jnp.mean((zq_s_f - z) ** 2)
                            + jnp.mean((zq_t_f - z) ** 2))
            + cos_loss)
    return zq_out, loss, ind_s, ind_t


# speedup vs baseline: 1.7478x; 1.7478x over previous
import jax
import jax.numpy as jnp
from jax.experimental import pallas as pl
from jax.experimental.pallas import tpu as pltpu

_N_E = 1024
_E_DIM = 64
_DEPTH = 6
_BM = 512


def _vq_block(zf_ref, sw_ref, tw_ref, zqs_ref, zqt_ref, inds_ref, indt_ref):
    x = zf_ref[...]
    col = jax.lax.broadcasted_iota(jnp.int32, (x.shape[0], _N_E), 1)
    sw = sw_ref[...]
    tw = tw_ref[...]
    sw_sq = jnp.sum(sw ** 2, axis=1)
    tw_sq = jnp.sum(tw ** 2, axis=1)
    sw_m2 = -2.0 * sw
    tw_m2 = -2.0 * tw
    res_s, zq_s = x, jnp.zeros_like(x)
    res_t, zq_t = x, jnp.zeros_like(x)
    for depth in range(_DEPTH):
        def step(residual, zq, cb, cb_m2, cb_sq, ind_ref):
            d = ((jnp.sum(residual ** 2, axis=1, keepdims=True) + cb_sq)
                 + residual @ cb_m2.T)
            dmin = jnp.min(d, axis=1, keepdims=True)
            mi = jnp.min(jnp.where(d <= dmin, col, _N_E), axis=1)
            onehot = (col == mi[:, None]).astype(jnp.float32)
            delta = jax.lax.dot_general(
                onehot, cb, (((1,), (0,)), ((), ())),
                preferred_element_type=jnp.float32)
            ind_ref[0, depth, :] = mi
            return residual - delta, zq + delta

        res_s, zq_s = step(res_s, zq_s, sw, sw_m2, sw_sq, inds_ref)
        res_t, zq_t = step(res_t, zq_t, tw, tw_m2, tw_sq, indt_ref)
    zqs_ref[...] = zq_s
    zqt_ref[...] = zq_t


def _cos_block(sw_ref, tw_ref, out_ref):
    sw = sw_ref[...]
    tw = tw_ref[...]
    sn = sw / (jnp.sqrt(jnp.sum(sw * sw, axis=1, keepdims=True)) + 1e-8)
    tn = tw / (jnp.sqrt(jnp.sum(tw * tw, axis=1, keepdims=True)) + 1e-8)
    m = jnp.dot(sn, tn.T, preferred_element_type=jnp.float32)
    out_ref[...] = (jnp.sum(m * m) / (_N_E * _N_E))[None, None]


def kernel(z, shared_w, task_w):
    b, c, h, w = z.shape
    ks = 2
    lh, lw = h - ks + 1, w - ks + 1
    pats = [z[:, :, i:i + lh, j:j + lw] for i in range(ks) for j in range(ks)]
    p = jnp.stack(pats, axis=2)
    zf = (p.reshape(b, c * ks * ks, lh * lw)
           .transpose(0, 2, 1)
           .reshape(-1, _E_DIM))
    n = zf.shape[0]
    nblk = (n + _BM - 1) // _BM
    npad = nblk * _BM
    zf_pad = jnp.pad(zf, ((0, npad - n), (0, 0)))

    zqs, zqt, inds_blk, indt_blk = pl.pallas_call(
        _vq_block,
        grid=(nblk,),
        compiler_params=pltpu.CompilerParams(
            dimension_semantics=("parallel",)),
        in_specs=[
            pl.BlockSpec((_BM, _E_DIM), lambda i: (i, 0)),
            pl.BlockSpec((_N_E, _E_DIM), lambda i: (0, 0)),
            pl.BlockSpec((_N_E, _E_DIM), lambda i: (0, 0)),
        ],
        out_specs=[
            pl.BlockSpec((_BM, _E_DIM), lambda i: (i, 0)),
            pl.BlockSpec((_BM, _E_DIM), lambda i: (i, 0)),
            pl.BlockSpec((1, _DEPTH, _BM), lambda i: (i, 0, 0)),
            pl.BlockSpec((1, _DEPTH, _BM), lambda i: (i, 0, 0)),
        ],
        out_shape=[
            jax.ShapeDtypeStruct((npad, _E_DIM), jnp.float32),
            jax.ShapeDtypeStruct((npad, _E_DIM), jnp.float32),
            jax.ShapeDtypeStruct((nblk, _DEPTH, _BM), jnp.int32),
            jax.ShapeDtypeStruct((nblk, _DEPTH, _BM), jnp.int32),
        ],
    )(zf_pad, shared_w, task_w)

    ind_s = (inds_blk.transpose(0, 2, 1).reshape(npad, _DEPTH)[:n]
             .reshape(b, lh, lw, _DEPTH))
    ind_t = (indt_blk.transpose(0, 2, 1).reshape(npad, _DEPTH)[:n]
             .reshape(b, lh, lw, _DEPTH))

    ch = jnp.where((jnp.arange(h) == 0) | (jnp.arange(h) == h - 1), 1.0, 2.0)
    cw = jnp.where((jnp.arange(w) == 0) | (jnp.arange(w) == w - 1), 1.0, 2.0)
    cnt = ch[:, None] * cw[None, :]

    def fold(zq_flat):
        zq = zq_flat[:n].reshape(b, lh, lw, c, ks * ks)
        out = jnp.zeros((b, c, h, w), jnp.float32)
        idx = 0
        for i in range(ks):
            for j in range(ks):
                out = out.at[:, :, i:i + lh, j:j + lw].add(
                    zq[..., idx].transpose(0, 3, 1, 2))
                idx += 1
        return out / cnt

    zq_s_f = fold(zqs)
    zq_t_f = fold(zqt)
    zq_out = 0.5 * (zq_s_f + zq_t_f)

    cos_loss = pl.pallas_call(
        _cos_block,
        out_shape=jax.ShapeDtypeStruct((1, 1), jnp.float32),
    )(shared_w, task_w)[0, 0]

    beta = 0.25
    loss = ((1.0 + beta) * (jnp.mean((zq_s_f - z) ** 2)
                            + jnp.mean((zq_t_f - z) ** 2))
            + cos_loss)
    return zq_out, loss, ind_s, ind_t
